# baseline (speedup 1.0000x reference)
import functools

import jax
import jax.numpy as jnp
from jax import lax
from jax.experimental import pallas as pl
from jax.experimental.pallas import tpu as pltpu
from jax.experimental.pallas import tpu_sc as plsc

D = 64
L = 16
CB = 16384
SCB = 512
SC_BLOCKS = 20
SC_COLS = 32 * SC_BLOCKS * SCB
assert SC_COLS % CB == 0


def _matvec_body(b_ref, ut_ref, it_ref, wu_ref, wi_ref, pu_ref, pi_ref):
    pu_ref[...] = jnp.sum(ut_ref[...] * wu_ref[...], axis=0) + b_ref[0]
    pi_ref[...] = jnp.sum(it_ref[...] * wi_ref[...], axis=0)


def _tc_score_vectors(ut_t, it_t, wu, wi, fc_b):
    n = ut_t.shape[1]
    blk0 = SC_COLS // CB
    grid = (n - SC_COLS + CB - 1) // CB
    return pl.pallas_call(
        _matvec_body,
        grid=(grid,),
        in_specs=[
            pl.BlockSpec(memory_space=pltpu.SMEM),
            pl.BlockSpec((D, CB), lambda i: (0, blk0 + i)),
            pl.BlockSpec((D, CB), lambda i: (0, blk0 + i)),
            pl.BlockSpec((D, 1), lambda i: (0, 0)),
            pl.BlockSpec((D, 1), lambda i: (0, 0)),
        ],
        out_specs=[
            pl.BlockSpec((CB,), lambda i: (blk0 + i,)),
            pl.BlockSpec((CB,), lambda i: (blk0 + i,)),
        ],
        out_shape=[
            jax.ShapeDtypeStruct((n,), jnp.float32),
            jax.ShapeDtypeStruct((n,), jnp.float32),
        ],
    )(fc_b, ut_t, it_t, wu, wi)


def _sc_score_vectors(ut_t, it_t, wsp):
    info = plsc.get_sparse_core_info()
    nc = info.num_cores
    span = SC_BLOCKS * SCB
    mesh = plsc.VectorSubcoreMesh(core_axis_name="c", subcore_axis_name="s")

    @functools.partial(
        pl.kernel,
        mesh=mesh,
        out_type=[
            jax.ShapeDtypeStruct((SC_COLS,), jnp.float32),
            jax.ShapeDtypeStruct((SC_COLS,), jnp.float32),
        ],
        compiler_params=pltpu.CompilerParams(
            needs_layout_passes=False, use_tc_tiling_on_sc=True
        ),
        scratch_types=[
            pltpu.VMEM((D, SCB), jnp.float32),
            pltpu.VMEM((D, SCB), jnp.float32),
            pltpu.VMEM((2 * D + 8, 128), jnp.float32),
            pltpu.VMEM((SCB,), jnp.float32),
            pltpu.VMEM((SCB,), jnp.float32),
        ],
    )
    def sc_matvec(ut_hbm, it_hbm, wsp_hbm, pu_hbm, pi_hbm,
                  tu_v, ti_v, wsp_v, pu_v, pi_v):
        wid = lax.axis_index("s") * nc + lax.axis_index("c")
        base = wid * span
        pltpu.sync_copy(wsp_hbm, wsp_v)
        wu = [wsp_v[e, pl.ds(0, L)] for e in range(D)]
        wi = [wsp_v[D + e, pl.ds(0, L)] for e in range(D)]
        bias = wsp_v[2 * D, pl.ds(0, L)]

        def block_body(b, carry):
            c0 = pl.multiple_of(base + b * SCB, SCB)
            pltpu.sync_copy(ut_hbm.at[:, pl.ds(c0, SCB)], tu_v)
            pltpu.sync_copy(it_hbm.at[:, pl.ds(c0, SCB)], ti_v)

            def col_body(g, carry2):
                cs = pl.multiple_of(g * L, L)
                acc_u = tu_v[0, pl.ds(cs, L)] * wu[0]
                acc_i = ti_v[0, pl.ds(cs, L)] * wi[0]
                for e in range(1, D):
                    acc_u = acc_u + tu_v[e, pl.ds(cs, L)] * wu[e]
                    acc_i = acc_i + ti_v[e, pl.ds(cs, L)] * wi[e]
                pu_v[pl.ds(cs, L)] = acc_u + bias
                pi_v[pl.ds(cs, L)] = acc_i
                return carry2

            lax.fori_loop(0, SCB // L, col_body, 0)
            pltpu.sync_copy(pu_v, pu_hbm.at[pl.ds(c0, SCB)])
            pltpu.sync_copy(pi_v, pi_hbm.at[pl.ds(c0, SCB)])
            return carry

        lax.fori_loop(0, SC_BLOCKS, block_body, 0)

    return sc_matvec(ut_t, it_t, wsp)


def kernel(user_ids, item_ids, user_table, item_table, fc_w, fc_b):
    batch = user_ids.shape[0]
    info = plsc.get_sparse_core_info()
    nw = info.num_cores * info.num_subcores
    bpw = batch // nw

    ut_t = user_table.T
    it_t = item_table.T
    wu = fc_w[:D]
    wi = fc_w[D:]
    wsp = jnp.concatenate(
        [
            jnp.broadcast_to(wu, (D, 128)),
            jnp.broadcast_to(wi, (D, 128)),
            jnp.broadcast_to(fc_b.reshape(1, 1), (8, 128)),
        ]
    )

    pu_sc, pi_sc = _sc_score_vectors(ut_t, it_t, wsp)
    pu_tc, pi_tc = _tc_score_vectors(ut_t, it_t, wu, wi, fc_b)
    pu = jnp.concatenate([pu_sc, lax.slice(pu_tc, (SC_COLS,), (ut_t.shape[1],))])
    pi = jnp.concatenate([pi_sc, lax.slice(pi_tc, (SC_COLS,), (it_t.shape[1],))])

    mesh = plsc.VectorSubcoreMesh(core_axis_name="c", subcore_axis_name="s")

    @functools.partial(
        pl.kernel,
        mesh=mesh,
        out_type=jax.ShapeDtypeStruct((batch,), jnp.float32),
        compiler_params=pltpu.CompilerParams(
            needs_layout_passes=False, use_tc_tiling_on_sc=False
        ),
        scratch_types=[
            pltpu.VMEM((bpw,), jnp.int32),
            pltpu.VMEM((bpw,), jnp.int32),
            pltpu.VMEM((bpw,), jnp.float32),
            pltpu.VMEM((bpw,), jnp.float32),
            pltpu.VMEM((bpw,), jnp.float32),
            pltpu.SemaphoreType.DMA,
            pltpu.SemaphoreType.DMA,
        ],
    )
    def sc_gather(uid_hbm, iid_hbm, pu_hbm, pi_hbm, out_hbm,
                  uidx_v, iidx_v, puv_v, piv_v, out_v, sem_u, sem_i):
        wid = lax.axis_index("s") * info.num_cores + lax.axis_index("c")
        base = wid * bpw
        pltpu.sync_copy(uid_hbm.at[pl.ds(base, bpw)], uidx_v)
        pltpu.sync_copy(iid_hbm.at[pl.ds(base, bpw)], iidx_v)
        cu = pltpu.async_copy(pu_hbm.at[uidx_v], puv_v, sem_u)
        ci = pltpu.async_copy(pi_hbm.at[iidx_v], piv_v, sem_i)
        cu.wait()
        ci.wait()
        for g in range(bpw // L):
            out_v[pl.ds(g * L, L)] = (
                puv_v[pl.ds(g * L, L)] + piv_v[pl.ds(g * L, L)]
            )
        pltpu.sync_copy(out_v, out_hbm.at[pl.ds(base, bpw)])

    return sc_gather(user_ids, item_ids, pu, pi)


# speedup vs baseline: 1.2036x; 1.2036x over previous
import functools

import jax
import jax.numpy as jnp
from jax import lax
from jax.experimental import pallas as pl
from jax.experimental.pallas import tpu as pltpu
from jax.experimental.pallas import tpu_sc as plsc

D = 64
L = 16
CB = 16384
SCB = 128
SC_BLOCKS = 80
SC_COLS = 32 * SC_BLOCKS * SCB
assert SC_COLS % CB == 0


def _matvec_body(b_ref, ut_ref, it_ref, wu_ref, wi_ref, pu_ref, pi_ref):
    pu_ref[...] = jnp.sum(ut_ref[...] * wu_ref[...], axis=0) + b_ref[0]
    pi_ref[...] = jnp.sum(it_ref[...] * wi_ref[...], axis=0)


def _tc_score_vectors(ut_t, it_t, wu, wi, fc_b):
    n = ut_t.shape[1]
    blk0 = SC_COLS // CB
    grid = (n - SC_COLS + CB - 1) // CB
    return pl.pallas_call(
        _matvec_body,
        grid=(grid,),
        in_specs=[
            pl.BlockSpec(memory_space=pltpu.SMEM),
            pl.BlockSpec((D, CB), lambda i: (0, blk0 + i)),
            pl.BlockSpec((D, CB), lambda i: (0, blk0 + i)),
            pl.BlockSpec((D, 1), lambda i: (0, 0)),
            pl.BlockSpec((D, 1), lambda i: (0, 0)),
        ],
        out_specs=[
            pl.BlockSpec((CB,), lambda i: (blk0 + i,)),
            pl.BlockSpec((CB,), lambda i: (blk0 + i,)),
        ],
        out_shape=[
            jax.ShapeDtypeStruct((n,), jnp.float32),
            jax.ShapeDtypeStruct((n,), jnp.float32),
        ],
    )(fc_b, ut_t, it_t, wu, wi)


def _sc_score_vectors(ut_t, it_t, wsp):
    info = plsc.get_sparse_core_info()
    nc = info.num_cores
    span = SC_BLOCKS * SCB
    mesh = plsc.VectorSubcoreMesh(core_axis_name="c", subcore_axis_name="s")

    @functools.partial(
        pl.kernel,
        mesh=mesh,
        out_type=[
            jax.ShapeDtypeStruct((SC_COLS,), jnp.float32),
            jax.ShapeDtypeStruct((SC_COLS,), jnp.float32),
        ],
        compiler_params=pltpu.CompilerParams(
            needs_layout_passes=False, use_tc_tiling_on_sc=True
        ),
        scratch_types=[
            pltpu.VMEM((2, D, SCB), jnp.float32),
            pltpu.VMEM((2, D, SCB), jnp.float32),
            pltpu.VMEM((2 * D + 8, 128), jnp.float32),
            pltpu.VMEM((SCB,), jnp.float32),
            pltpu.VMEM((SCB,), jnp.float32),
            pltpu.SemaphoreType.DMA,
            pltpu.SemaphoreType.DMA,
        ],
    )
    def sc_matvec(ut_hbm, it_hbm, wsp_hbm, pu_hbm, pi_hbm,
                  tu_v, ti_v, wsp_v, pu_v, pi_v, sem_u, sem_i):
        wid = lax.axis_index("s") * nc + lax.axis_index("c")
        base = wid * span
        pltpu.sync_copy(wsp_hbm, wsp_v)
        wu = [wsp_v[e, pl.ds(0, L)] for e in range(D)]
        wi = [wsp_v[D + e, pl.ds(0, L)] for e in range(D)]
        bias = wsp_v[2 * D, pl.ds(0, L)]

        def start(b, slot):
            c0 = pl.multiple_of(base + b * SCB, SCB)
            pltpu.async_copy(ut_hbm.at[:, pl.ds(c0, SCB)], tu_v.at[slot], sem_u)
            pltpu.async_copy(it_hbm.at[:, pl.ds(c0, SCB)], ti_v.at[slot], sem_i)

        def wait(slot):
            pltpu.make_async_copy(
                ut_hbm.at[:, pl.ds(0, SCB)], tu_v.at[slot], sem_u
            ).wait()
            pltpu.make_async_copy(
                it_hbm.at[:, pl.ds(0, SCB)], ti_v.at[slot], sem_i
            ).wait()

        start(0, 0)

        def block_body(b, carry):
            slot = lax.rem(b, 2)
            wait(slot)

            @pl.when(b + 1 < SC_BLOCKS)
            def _():
                start(b + 1, lax.rem(b + 1, 2))

            for k in range(SCB // L):
                acc_u = tu_v[slot, 0, pl.ds(k * L, L)] * wu[0]
                acc_i = ti_v[slot, 0, pl.ds(k * L, L)] * wi[0]
                for e in range(1, D):
                    acc_u = acc_u + tu_v[slot, e, pl.ds(k * L, L)] * wu[e]
                    acc_i = acc_i + ti_v[slot, e, pl.ds(k * L, L)] * wi[e]
                pu_v[pl.ds(k * L, L)] = acc_u + bias
                pi_v[pl.ds(k * L, L)] = acc_i

            c0 = pl.multiple_of(base + b * SCB, SCB)
            pltpu.sync_copy(pu_v, pu_hbm.at[pl.ds(c0, SCB)])
            pltpu.sync_copy(pi_v, pi_hbm.at[pl.ds(c0, SCB)])
            return carry

        lax.fori_loop(0, SC_BLOCKS, block_body, 0)

    return sc_matvec(ut_t, it_t, wsp)


def kernel(user_ids, item_ids, user_table, item_table, fc_w, fc_b):
    batch = user_ids.shape[0]
    info = plsc.get_sparse_core_info()
    nw = info.num_cores * info.num_subcores
    bpw = batch // nw

    ut_t = user_table.T
    it_t = item_table.T
    wu = fc_w[:D]
    wi = fc_w[D:]
    wsp = jnp.concatenate(
        [
            jnp.broadcast_to(wu, (D, 128)),
            jnp.broadcast_to(wi, (D, 128)),
            jnp.broadcast_to(fc_b.reshape(1, 1), (8, 128)),
        ]
    )

    pu_sc, pi_sc = _sc_score_vectors(ut_t, it_t, wsp)
    pu_tc, pi_tc = _tc_score_vectors(ut_t, it_t, wu, wi, fc_b)
    pu = jnp.concatenate([pu_sc, lax.slice(pu_tc, (SC_COLS,), (ut_t.shape[1],))])
    pi = jnp.concatenate([pi_sc, lax.slice(pi_tc, (SC_COLS,), (it_t.shape[1],))])

    mesh = plsc.VectorSubcoreMesh(core_axis_name="c", subcore_axis_name="s")

    @functools.partial(
        pl.kernel,
        mesh=mesh,
        out_type=jax.ShapeDtypeStruct((batch,), jnp.float32),
        compiler_params=pltpu.CompilerParams(
            needs_layout_passes=False, use_tc_tiling_on_sc=False
        ),
        scratch_types=[
            pltpu.VMEM((bpw,), jnp.int32),
            pltpu.VMEM((bpw,), jnp.int32),
            pltpu.VMEM((bpw,), jnp.float32),
            pltpu.VMEM((bpw,), jnp.float32),
            pltpu.VMEM((bpw,), jnp.float32),
            pltpu.SemaphoreType.DMA,
            pltpu.SemaphoreType.DMA,
        ],
    )
    def sc_gather(uid_hbm, iid_hbm, pu_hbm, pi_hbm, out_hbm,
                  uidx_v, iidx_v, puv_v, piv_v, out_v, sem_u, sem_i):
        wid = lax.axis_index("s") * info.num_cores + lax.axis_index("c")
        base = wid * bpw
        pltpu.sync_copy(uid_hbm.at[pl.ds(base, bpw)], uidx_v)
        pltpu.sync_copy(iid_hbm.at[pl.ds(base, bpw)], iidx_v)
        cu = pltpu.async_copy(pu_hbm.at[uidx_v], puv_v, sem_u)
        ci = pltpu.async_copy(pi_hbm.at[iidx_v], piv_v, sem_i)
        cu.wait()
        ci.wait()
        for g in range(bpw // L):
            out_v[pl.ds(g * L, L)] = (
                puv_v[pl.ds(g * L, L)] + piv_v[pl.ds(g * L, L)]
            )
        pltpu.sync_copy(out_v, out_hbm.at[pl.ds(base, bpw)])

    return sc_gather(user_ids, item_ids, pu, pi)


# speedup vs baseline: 1.6789x; 1.3949x over previous
import functools

import jax
import jax.numpy as jnp
from jax import lax
from jax.experimental import pallas as pl
from jax.experimental.pallas import tpu as pltpu
from jax.experimental.pallas import tpu_sc as plsc

D = 64
L = 16
CB = 32768


def _matvec_body(b_ref, ut_ref, it_ref, wu_ref, wi_ref, pu_ref, pi_ref):
    pu_ref[...] = jnp.sum(ut_ref[...] * wu_ref[...], axis=0) + b_ref[0]
    pi_ref[...] = jnp.sum(it_ref[...] * wi_ref[...], axis=0)


def _score_vectors(ut_t, it_t, wu, wi, fc_b):
    n = ut_t.shape[1]
    grid = (n + CB - 1) // CB
    return pl.pallas_call(
        _matvec_body,
        grid=(grid,),
        in_specs=[
            pl.BlockSpec(memory_space=pltpu.SMEM),
            pl.BlockSpec((D, CB), lambda i: (0, i)),
            pl.BlockSpec((D, CB), lambda i: (0, i)),
            pl.BlockSpec((D, 1), lambda i: (0, 0)),
            pl.BlockSpec((D, 1), lambda i: (0, 0)),
        ],
        out_specs=[
            pl.BlockSpec((CB,), lambda i: (i,)),
            pl.BlockSpec((CB,), lambda i: (i,)),
        ],
        out_shape=[
            jax.ShapeDtypeStruct((n,), jnp.float32),
            jax.ShapeDtypeStruct((n,), jnp.float32),
        ],
    )(fc_b, ut_t, it_t, wu, wi)


def kernel(user_ids, item_ids, user_table, item_table, fc_w, fc_b):
    batch = user_ids.shape[0]
    info = plsc.get_sparse_core_info()
    nw = info.num_cores * info.num_subcores
    bpw = batch // nw

    ut_t = user_table.T
    it_t = item_table.T
    wu = fc_w[:D]
    wi = fc_w[D:]

    pu, pi = _score_vectors(ut_t, it_t, wu, wi, fc_b)

    mesh = plsc.VectorSubcoreMesh(core_axis_name="c", subcore_axis_name="s")

    @functools.partial(
        pl.kernel,
        mesh=mesh,
        out_type=jax.ShapeDtypeStruct((batch,), jnp.float32),
        compiler_params=pltpu.CompilerParams(
            needs_layout_passes=False, use_tc_tiling_on_sc=False
        ),
        scratch_types=[
            pltpu.VMEM((bpw,), jnp.int32),
            pltpu.VMEM((bpw,), jnp.int32),
            pltpu.VMEM((bpw,), jnp.float32),
            pltpu.VMEM((bpw,), jnp.float32),
            pltpu.VMEM((bpw,), jnp.float32),
            pltpu.SemaphoreType.DMA,
            pltpu.SemaphoreType.DMA,
        ],
    )
    def sc_gather(uid_hbm, iid_hbm, pu_hbm, pi_hbm, out_hbm,
                  uidx_v, iidx_v, puv_v, piv_v, out_v, sem_u, sem_i):
        wid = lax.axis_index("s") * info.num_cores + lax.axis_index("c")
        base = wid * bpw
        pltpu.sync_copy(uid_hbm.at[pl.ds(base, bpw)], uidx_v)
        pltpu.sync_copy(iid_hbm.at[pl.ds(base, bpw)], iidx_v)
        cu = pltpu.async_copy(pu_hbm.at[uidx_v], puv_v, sem_u)
        ci = pltpu.async_copy(pi_hbm.at[iidx_v], piv_v, sem_i)
        cu.wait()
        ci.wait()
        for g in range(bpw // L):
            out_v[pl.ds(g * L, L)] = (
                puv_v[pl.ds(g * L, L)] + piv_v[pl.ds(g * L, L)]
            )
        pltpu.sync_copy(out_v, out_hbm.at[pl.ds(base, bpw)])

    return sc_gather(user_ids, item_ids, pu, pi)
